# baseline (speedup 1.0000x reference)
import functools

import jax
import jax.numpy as jnp
from jax import lax
from jax.experimental import pallas as pl
from jax.experimental.pallas import tpu as pltpu
from jax.experimental.pallas import tpu_sc as plsc

_NC = 2
_NS = 16
_NW = _NC * _NS

_OUT_DIM = 64
_CHUNK = 256
_NBUF = 4


def _make_gather(n_rows: int):
    assert n_rows % (_NW * _CHUNK * _NBUF) == 0
    chunks_per_w = n_rows // (_NW * _CHUNK)
    n_groups = chunks_per_w // _NBUF

    mesh = plsc.VectorSubcoreMesh(core_axis_name="c", subcore_axis_name="s")
    scratch = [pltpu.VMEM((chunks_per_w, _CHUNK), jnp.int32)]
    scratch += [pltpu.VMEM((_CHUNK, _OUT_DIM), jnp.float32)] * _NBUF
    scratch += [pltpu.SemaphoreType.DMA] * (2 * _NBUF)

    @functools.partial(
        pl.kernel,
        out_type=jax.ShapeDtypeStruct((n_rows, _OUT_DIM), jnp.float32),
        mesh=mesh,
        scratch_types=scratch,
        compiler_params=pltpu.CompilerParams(use_tc_tiling_on_sc=False),
    )
    def gather(table_hbm, idx_hbm, out_hbm, idx_v, *bufs):
        rows = bufs[:_NBUF]
        gsem = bufs[_NBUF : 2 * _NBUF]
        ssem = bufs[2 * _NBUF :]
        wid = lax.axis_index("s") * _NC + lax.axis_index("c")
        cbase = wid * chunks_per_w
        pltpu.sync_copy(idx_hbm.at[pl.ds(cbase, chunks_per_w)], idx_v)

        def gather_copy(j, b):
            return pltpu.make_async_copy(
                table_hbm.at[idx_v.at[j]], rows[b], gsem[b]
            )

        def store_copy(j, b):
            return pltpu.make_async_copy(
                rows[b], out_hbm.at[pl.ds((cbase + j) * _CHUNK, _CHUNK)], ssem[b]
            )

        for b in range(_NBUF):
            gather_copy(b, b).start()

        def group(g, carry):
            j0 = g * _NBUF
            for b in range(_NBUF):
                gather_copy(j0 + b, b).wait()
                store_copy(j0 + b, b).start()
            for b in range(_NBUF):
                store_copy(j0 + b, b).wait()
                gather_copy(j0 + _NBUF + b, b).start()
            return carry

        lax.fori_loop(0, n_groups - 1, group, 0)

        j0 = (n_groups - 1) * _NBUF
        for b in range(_NBUF):
            gather_copy(j0 + b, b).wait()
            store_copy(j0 + b, b).start()
        for b in range(_NBUF):
            store_copy(j0 + b, b).wait()

    return gather


def kernel(input_ids, table):
    batch, hist = input_ids.shape
    n = batch * hist
    idx2d = input_ids.reshape(n // _CHUNK, _CHUNK).astype(jnp.int32)
    out = _make_gather(n)(table, idx2d)
    return out.reshape(batch, hist, _OUT_DIM)


# speedup vs baseline: 1.0021x; 1.0021x over previous
import functools

import jax
import jax.numpy as jnp
from jax import lax
from jax.experimental import pallas as pl
from jax.experimental.pallas import tpu as pltpu
from jax.experimental.pallas import tpu_sc as plsc

_NC = 2
_NS = 16
_NW = _NC * _NS

_OUT_DIM = 64
_CHUNK = 128
_K = 4


def _make_gather(n_rows: int):
    assert n_rows % (_NW * _CHUNK * 2 * _K) == 0
    chunks_per_w = n_rows // (_NW * _CHUNK)
    n_pairs = chunks_per_w // (2 * _K)

    mesh = plsc.VectorSubcoreMesh(core_axis_name="c", subcore_axis_name="s")
    nbuf = 2 * _K
    scratch = [pltpu.VMEM((chunks_per_w, _CHUNK), jnp.int32)]
    scratch += [pltpu.VMEM((_CHUNK, _OUT_DIM), jnp.float32)] * nbuf
    scratch += [pltpu.SemaphoreType.DMA] * (2 * nbuf)

    @functools.partial(
        pl.kernel,
        out_type=jax.ShapeDtypeStruct((n_rows, _OUT_DIM), jnp.float32),
        mesh=mesh,
        scratch_types=scratch,
        compiler_params=pltpu.CompilerParams(use_tc_tiling_on_sc=False),
    )
    def gather(table_hbm, idx_hbm, out_hbm, idx_v, *bufs):
        rows = bufs[:nbuf]
        gsem = bufs[nbuf : 2 * nbuf]
        ssem = bufs[2 * nbuf :]
        wid = lax.axis_index("s") * _NC + lax.axis_index("c")
        cbase = wid * chunks_per_w
        pltpu.sync_copy(idx_hbm.at[pl.ds(cbase, chunks_per_w)], idx_v)

        def gather_copy(j, b):
            return pltpu.make_async_copy(
                table_hbm.at[idx_v.at[j]], rows[b], gsem[b]
            )

        def store_copy(j, b):
            return pltpu.make_async_copy(
                rows[b], out_hbm.at[pl.ds((cbase + j) * _CHUNK, _CHUNK)], ssem[b]
            )

        for s in range(_K):
            gather_copy(s, s).start()

        def pair(t, carry):
            g0 = 2 * t
            for s in range(_K):
                b = _K + s

                @pl.when(t > 0)
                def _():
                    store_copy((g0 - 1) * _K + s, b).wait()

                gather_copy((g0 + 1) * _K + s, b).start()
            for s in range(_K):
                gather_copy(g0 * _K + s, s).wait()
                store_copy(g0 * _K + s, s).start()
            for s in range(_K):
                store_copy(g0 * _K + s, s).wait()

                @pl.when(t < n_pairs - 1)
                def _():
                    gather_copy((g0 + 2) * _K + s, s).start()
            for s in range(_K):
                b = _K + s
                gather_copy((g0 + 1) * _K + s, b).wait()
                store_copy((g0 + 1) * _K + s, b).start()
            return carry

        lax.fori_loop(0, n_pairs, pair, 0)

        for s in range(_K):
            store_copy((2 * n_pairs - 1) * _K + s, _K + s).wait()

    return gather


def kernel(input_ids, table):
    batch, hist = input_ids.shape
    n = batch * hist
    idx2d = input_ids.reshape(n // _CHUNK, _CHUNK).astype(jnp.int32)
    out = _make_gather(n)(table, idx2d)
    return out.reshape(batch, hist, _OUT_DIM)
